# Initial kernel scaffold; baseline (speedup 1.0000x reference)
import jax
import jax.numpy as jnp
from jax.experimental import pallas as pl


def kernel(X, Edge_Index, W, b):
    raise NotImplementedError("write your pallas kernel here")


# speedup vs baseline: 32.0139x; 32.0139x over previous
import dataclasses
import functools

import jax
import jax.numpy as jnp
from jax import lax
from jax.experimental import pallas as pl
from jax.experimental.pallas import tpu as pltpu
from jax.experimental.pallas import tpu_sc as plsc

NC = 2
NS = 16
NW = NC * NS
LANES = 16
CHUNK = 125

_sc_mesh = functools.partial(
    plsc.VectorSubcoreMesh, core_axis_name="c", subcore_axis_name="s"
)


def _sc_params():
    cp = pltpu.CompilerParams()
    fields = pltpu.CompilerParams.__dataclass_fields__
    if "needs_layout_passes" in fields:
        cp = dataclasses.replace(cp, needs_layout_passes=False)
    if "use_tc_tiling_on_sc" in fields:
        cp = dataclasses.replace(cp, use_tc_tiling_on_sc=False)
    return cp


def _sc_degree(dst_rows, n):
    nw, epw = dst_rows.shape

    @pl.kernel(
        out_type=jax.ShapeDtypeStruct((nw, n), jnp.float32),
        mesh=_sc_mesh(),
        scratch_types=[
            pltpu.VMEM((epw,), jnp.int32),
            pltpu.VMEM((n,), jnp.float32),
            pltpu.SemaphoreType.DMA,
        ],
        compiler_params=_sc_params(),
    )
    def deg_kernel(dst_hbm, hist_hbm, dst_v, hist_v, sem):
        w = lax.axis_index("c") * NS + lax.axis_index("s")
        pltpu.async_copy(dst_hbm.at[w], dst_v, sem).wait()

        zeros = jnp.zeros((LANES,), jnp.float32)

        @pl.loop(0, n, step=LANES)
        def _(i):
            hist_v[pl.ds(i, LANES)] = zeros

        ones = jnp.ones((LANES,), jnp.float32)

        @pl.loop(0, epw, step=LANES)
        def _(i):
            idx = dst_v[pl.ds(i, LANES)]
            plsc.addupdate_scatter(hist_v, [idx], ones)

        pltpu.async_copy(hist_v, hist_hbm.at[w], sem).wait()

    return deg_kernel(dst_rows)


def _tc_transform(X, W, b2, hists, block_rows):
    n, d_in = X.shape
    d_hid = W.shape[1]
    nw = hists.shape[0]
    grid = n // block_rows
    hists = hists.reshape(nw, grid, block_rows).swapaxes(0, 1)

    def body(x_ref, w_ref, b_ref, h_ref, g_ref, dinv_ref):
        deg = jnp.sum(h_ref[0], axis=0) + 1.0
        dinv = lax.rsqrt(deg)
        h = (
            jnp.dot(x_ref[...], w_ref[...], preferred_element_type=jnp.float32)
            + b_ref[...]
        )
        g_ref[...] = h * dinv[:, None]
        dinv_ref[...] = dinv[:, None]

    return pl.pallas_call(
        body,
        grid=(grid,),
        in_specs=[
            pl.BlockSpec((block_rows, d_in), lambda i: (i, 0)),
            pl.BlockSpec((d_in, d_hid), lambda i: (0, 0)),
            pl.BlockSpec((1, d_hid), lambda i: (0, 0)),
            pl.BlockSpec((1, nw, block_rows), lambda i: (i, 0, 0)),
        ],
        out_specs=[
            pl.BlockSpec((block_rows, d_hid), lambda i: (i, 0)),
            pl.BlockSpec((block_rows, 1), lambda i: (i, 0)),
        ],
        out_shape=[
            jax.ShapeDtypeStruct((n, d_hid), jnp.float32),
            jax.ShapeDtypeStruct((n, 1), jnp.float32),
        ],
    )(X, W, b2, hists)


def _sc_scatter(g, src_c, dst_c):
    n, d = g.shape
    nw, nch, chunk = src_c.shape
    rows_per_tile = n // NS
    q_steps = rows_per_tile // chunk

    @pl.kernel(
        out_type=jax.ShapeDtypeStruct((NC, n, d), jnp.float32),
        mesh=_sc_mesh(),
        scratch_types=[
            pltpu.VMEM((nch, chunk), jnp.int32),
            pltpu.VMEM((nch, chunk), jnp.int32),
            pltpu.VMEM((chunk, d), jnp.float32),
            pltpu.VMEM_SHARED((n, d), jnp.float32),
            pltpu.SemaphoreType.DMA,
        ],
        compiler_params=_sc_params(),
    )
    def scat_kernel(g_hbm, si_hbm, di_hbm, out_hbm, si_v, di_v, rows_v, acc_sh, sem):
        c = lax.axis_index("c")
        s = lax.axis_index("s")
        w = c * NS + s

        pltpu.async_copy(si_hbm.at[w], si_v, sem).wait()
        pltpu.async_copy(di_hbm.at[w], di_v, sem).wait()

        zeros = jnp.zeros((LANES,), jnp.float32)

        @pl.loop(0, chunk)
        def _(r):
            @pl.loop(0, d, step=LANES)
            def _(k):
                rows_v[r, pl.ds(k, LANES)] = zeros

        @pl.loop(0, q_steps)
        def _(q):
            pltpu.sync_copy(
                rows_v, acc_sh.at[pl.ds(s * rows_per_tile + q * chunk, chunk)]
            )

        plsc.subcore_barrier()

        @pl.loop(0, nch)
        def _(j):
            pltpu.async_copy(g_hbm.at[si_v.at[j]], rows_v, sem).wait()
            pltpu.sync_copy(rows_v, acc_sh.at[di_v.at[j]], add=True)

        plsc.subcore_barrier()

        @pl.loop(0, q_steps)
        def _(q):
            base = s * rows_per_tile + q * chunk
            pltpu.sync_copy(acc_sh.at[pl.ds(base, chunk)], rows_v)
            pltpu.sync_copy(rows_v, out_hbm.at[c, pl.ds(base, chunk)])

    return scat_kernel(g, src_c, dst_c)


def _tc_logsoftmax(acc, g, dinv, block_rows):
    n, d = g.shape
    grid = n // block_rows

    def body(a_ref, g_ref, dinv_ref, o_ref):
        z = dinv_ref[...] * (a_ref[0] + a_ref[1] + g_ref[...])
        m = jnp.max(z, axis=1, keepdims=True)
        e = jnp.exp(z - m)
        ssum = jnp.sum(e, axis=1, keepdims=True)
        o_ref[...] = z - m - jnp.log(ssum)

    return pl.pallas_call(
        body,
        grid=(grid,),
        in_specs=[
            pl.BlockSpec((2, block_rows, d), lambda i: (0, i, 0)),
            pl.BlockSpec((block_rows, d), lambda i: (i, 0)),
            pl.BlockSpec((block_rows, 1), lambda i: (i, 0)),
        ],
        out_specs=pl.BlockSpec((block_rows, d), lambda i: (i, 0)),
        out_shape=jax.ShapeDtypeStruct((n, d), jnp.float32),
    )(acc, g, dinv)


def kernel(X, Edge_Index, W, b):
    n, d_in = X.shape
    e = Edge_Index.shape[1]
    d_hid = W.shape[1]

    epw = e // NW
    nch = epw // CHUNK

    src_c = Edge_Index[0].reshape(NW, nch, CHUNK)
    dst_c = Edge_Index[1].reshape(NW, nch, CHUNK)
    dst_rows = Edge_Index[1].reshape(NW, epw)

    hists = _sc_degree(dst_rows, n)
    g, dinv = _tc_transform(X, W, b.reshape(1, d_hid), hists, block_rows=1000)
    acc = _sc_scatter(g, src_c, dst_c)
    return _tc_logsoftmax(acc, g, dinv, block_rows=1000)


# speedup vs baseline: 42.4032x; 1.3245x over previous
import dataclasses
import functools

import jax
import jax.numpy as jnp
from jax import lax
from jax.experimental import pallas as pl
from jax.experimental.pallas import tpu as pltpu
from jax.experimental.pallas import tpu_sc as plsc

NC = 2
NS = 16
NW = NC * NS
LANES = 16
CHUNK = 100

_sc_mesh = functools.partial(
    plsc.VectorSubcoreMesh, core_axis_name="c", subcore_axis_name="s"
)


def _sc_params():
    cp = pltpu.CompilerParams()
    fields = pltpu.CompilerParams.__dataclass_fields__
    if "needs_layout_passes" in fields:
        cp = dataclasses.replace(cp, needs_layout_passes=False)
    if "use_tc_tiling_on_sc" in fields:
        cp = dataclasses.replace(cp, use_tc_tiling_on_sc=False)
    return cp


def _sc_degree(dst_rows, n):
    nw, epw = dst_rows.shape

    @pl.kernel(
        out_type=jax.ShapeDtypeStruct((nw, n), jnp.float32),
        mesh=_sc_mesh(),
        scratch_types=[
            pltpu.VMEM((epw,), jnp.int32),
            pltpu.VMEM((n,), jnp.float32),
            pltpu.SemaphoreType.DMA,
        ],
        compiler_params=_sc_params(),
    )
    def deg_kernel(dst_hbm, hist_hbm, dst_v, hist_v, sem):
        w = lax.axis_index("c") * NS + lax.axis_index("s")
        pltpu.async_copy(dst_hbm.at[w], dst_v, sem).wait()

        zeros = jnp.zeros((LANES,), jnp.float32)

        @pl.loop(0, n, step=LANES)
        def _(i):
            hist_v[pl.ds(i, LANES)] = zeros

        ones = jnp.ones((LANES,), jnp.float32)

        @pl.loop(0, epw, step=LANES)
        def _(i):
            idx = dst_v[pl.ds(i, LANES)]
            plsc.addupdate_scatter(hist_v, [idx], ones)

        pltpu.async_copy(hist_v, hist_hbm.at[w], sem).wait()

    return deg_kernel(dst_rows)


def _tc_transform(X, W, b2, hists, block_rows):
    n, d_in = X.shape
    d_hid = W.shape[1]
    nw = hists.shape[0]
    grid = n // block_rows
    hists = hists.reshape(nw, grid, block_rows).swapaxes(0, 1)

    def body(x_ref, w_ref, b_ref, h_ref, g_ref, dinv_ref):
        deg = jnp.sum(h_ref[0], axis=0) + 1.0
        dinv = lax.rsqrt(deg)
        h = (
            jnp.dot(x_ref[...], w_ref[...], preferred_element_type=jnp.float32)
            + b_ref[...]
        )
        g_ref[...] = h * dinv[:, None]
        dinv_ref[...] = dinv[:, None]

    return pl.pallas_call(
        body,
        grid=(grid,),
        in_specs=[
            pl.BlockSpec((block_rows, d_in), lambda i: (i, 0)),
            pl.BlockSpec((d_in, d_hid), lambda i: (0, 0)),
            pl.BlockSpec((1, d_hid), lambda i: (0, 0)),
            pl.BlockSpec((1, nw, block_rows), lambda i: (i, 0, 0)),
        ],
        out_specs=[
            pl.BlockSpec((block_rows, d_hid), lambda i: (i, 0)),
            pl.BlockSpec((block_rows, 1), lambda i: (i, 0)),
        ],
        out_shape=[
            jax.ShapeDtypeStruct((n, d_hid), jnp.float32),
            jax.ShapeDtypeStruct((n, 1), jnp.float32),
        ],
    )(X, W, b2, hists)


def _sc_scatter(g, zeros, src_c, dst_c):
    n, d = g.shape
    nw, nch, chunk = src_c.shape
    rows_per_tile = n // NS

    @pl.kernel(
        out_type=jax.ShapeDtypeStruct((NC, n, d), jnp.float32),
        mesh=_sc_mesh(),
        scratch_types=[
            pltpu.VMEM((nch, chunk), jnp.int32),
            pltpu.VMEM((nch, chunk), jnp.int32),
            pltpu.VMEM((chunk, d), jnp.float32),
            pltpu.VMEM((chunk, d), jnp.float32),
            pltpu.VMEM_SHARED((n, d), jnp.float32),
            pltpu.SemaphoreType.DMA,
            pltpu.SemaphoreType.DMA,
        ],
        compiler_params=_sc_params(),
    )
    def scat_kernel(
        g_hbm, z_hbm, si_hbm, di_hbm, out_hbm, si_v, di_v, rows0_v, rows1_v,
        acc_sh, sem0, sem1,
    ):
        c = lax.axis_index("c")
        s = lax.axis_index("s")
        w = c * NS + s

        pltpu.async_copy(si_hbm.at[w], si_v, sem0).wait()
        pltpu.async_copy(di_hbm.at[w], di_v, sem0).wait()

        tile_rows = pl.ds(s * rows_per_tile, rows_per_tile)

        @pl.when(c == 0)
        def _():
            pltpu.sync_copy(g_hbm.at[tile_rows], acc_sh.at[tile_rows])

        @pl.when(c != 0)
        def _():
            pltpu.sync_copy(z_hbm.at[tile_rows], acc_sh.at[tile_rows])

        plsc.subcore_barrier()

        def start(j, buf, sem):
            pltpu.async_copy(g_hbm.at[si_v.at[j]], buf, sem)

        def finish(j, buf, sem):
            pltpu.make_async_copy(g_hbm.at[si_v.at[j]], buf, sem).wait()
            pltpu.sync_copy(buf, acc_sh.at[di_v.at[j]], add=True)

        start(0, rows0_v, sem0)

        @pl.loop(0, nch - 2, step=2)
        def _(j):
            start(j + 1, rows1_v, sem1)
            finish(j, rows0_v, sem0)
            start(j + 2, rows0_v, sem0)
            finish(j + 1, rows1_v, sem1)

        start(nch - 1, rows1_v, sem1)
        finish(nch - 2, rows0_v, sem0)
        finish(nch - 1, rows1_v, sem1)

        plsc.subcore_barrier()

        pltpu.sync_copy(acc_sh.at[tile_rows], out_hbm.at[c, tile_rows])

    return scat_kernel(g, zeros, src_c, dst_c)


def _tc_logsoftmax(acc, dinv, block_rows):
    _, n, d = acc.shape
    grid = n // block_rows

    def body(a_ref, dinv_ref, o_ref):
        z = dinv_ref[...] * (a_ref[0] + a_ref[1])
        m = jnp.max(z, axis=1, keepdims=True)
        e = jnp.exp(z - m)
        ssum = jnp.sum(e, axis=1, keepdims=True)
        o_ref[...] = z - m - jnp.log(ssum)

    return pl.pallas_call(
        body,
        grid=(grid,),
        in_specs=[
            pl.BlockSpec((2, block_rows, d), lambda i: (0, i, 0)),
            pl.BlockSpec((block_rows, 1), lambda i: (i, 0)),
        ],
        out_specs=pl.BlockSpec((block_rows, d), lambda i: (i, 0)),
        out_shape=jax.ShapeDtypeStruct((n, d), jnp.float32),
    )(acc, dinv)


def kernel(X, Edge_Index, W, b):
    n, d_in = X.shape
    e = Edge_Index.shape[1]
    d_hid = W.shape[1]

    epw = e // NW
    nch = epw // CHUNK

    src_c = Edge_Index[0].reshape(NW, nch, CHUNK)
    dst_c = Edge_Index[1].reshape(NW, nch, CHUNK)
    dst_rows = Edge_Index[1].reshape(NW, epw)

    hists = _sc_degree(dst_rows, n)
    g, dinv = _tc_transform(X, W, b.reshape(1, d_hid), hists, block_rows=1000)
    zeros = jnp.zeros((n, d_hid), jnp.float32)
    acc = _sc_scatter(g, zeros, src_c, dst_c)
    return _tc_logsoftmax(acc, dinv, block_rows=1000)
